# baseline (speedup 1.0000x reference)
import functools

import jax
import jax.numpy as jnp
from jax import lax
from jax.experimental import pallas as pl
from jax.experimental.pallas import tpu as pltpu
from jax.experimental.pallas import tpu_sc as plsc

_NUM_NODES = 1000000
_DIM = 64
_BATCH = 16384
_NC, _NS, _L = 2, 16, 16
_NW = _NC * _NS
_BPW = _BATCH // _NW
_CHUNK = 128
_NCHUNK = _BPW // _CHUNK

_LN2 = 0.6931471805599453
_SQRT2 = 1.4142135623730951


def _neg_log(p):
    bits = plsc.bitcast(p, jnp.int32)
    e = lax.shift_right_logical(bits, 23) - 127
    m = plsc.bitcast(
        (bits & jnp.int32(0x007FFFFF)) | jnp.int32(0x3F800000), jnp.float32
    )
    big = m > jnp.float32(_SQRT2)
    m = jnp.where(big, m * jnp.float32(0.5), m)
    e = jnp.where(big, e + 1, e)
    s = (m - jnp.float32(1.0)) / (m + jnp.float32(1.0))
    s2 = s * s
    lnm = s * (jnp.float32(2.0) + s2 * (jnp.float32(2.0 / 3.0)
               + s2 * (jnp.float32(2.0 / 5.0) + s2 * jnp.float32(2.0 / 7.0))))
    return -(e.astype(jnp.float32) * jnp.float32(_LN2) + lnm)


def _body(table_hbm, idxi_hbm, idxj_hbm, out_hbm,
          idxi_v, idxj_v, rows_i, rows_j, out_v, sem):
    wid = lax.axis_index("s") * _NC + lax.axis_index("c")
    base = wid * _BPW

    pltpu.sync_copy(idxi_hbm.at[wid], idxi_v)
    pltpu.sync_copy(idxj_hbm.at[wid], idxj_v)

    copies = []
    for c in range(_NCHUNK):
        dst = pl.ds(c * _CHUNK, _CHUNK)
        copies.append(pltpu.async_copy(
            table_hbm.at[idxi_v.at[c]], rows_i.at[dst], sem))
        copies.append(pltpu.async_copy(
            table_hbm.at[idxj_v.at[c]], rows_j.at[dst], sem))
    for cp in copies:
        cp.wait()

    lanes = lax.iota(jnp.int32, _L)

    def group(g, carry):
        row_idx = lanes + g * _L
        acc = jnp.zeros((_L,), jnp.float32)
        for d in range(_DIM):
            col = jnp.full((_L,), d, jnp.int32)
            vi = plsc.load_gather(rows_i, [row_idx, col])
            vj = plsc.load_gather(rows_j, [row_idx, col])
            acc = acc + vi * vj
        p = jnp.float32(1.0) / (jnp.float32(1.0) + jnp.exp(-acc))
        p = jnp.maximum(p, jnp.float32(1e-8))
        out_v[pl.ds(g * _L, _L)] = _neg_log(p)
        return carry

    lax.fori_loop(0, _BPW // _L, group, None)
    pltpu.sync_copy(out_v, out_hbm.at[pl.ds(base, _BPW)])


@functools.partial(
    pl.kernel,
    out_type=jax.ShapeDtypeStruct((_BATCH,), jnp.float32),
    mesh=plsc.VectorSubcoreMesh(
        core_axis_name="c", subcore_axis_name="s",
        num_cores=_NC, num_subcores=_NS),
    scratch_types=[
        pltpu.VMEM((_NCHUNK, _CHUNK), jnp.int32),
        pltpu.VMEM((_NCHUNK, _CHUNK), jnp.int32),
        pltpu.VMEM((_BPW, _DIM), jnp.float32),
        pltpu.VMEM((_BPW, _DIM), jnp.float32),
        pltpu.VMEM((_BPW,), jnp.float32),
        pltpu.SemaphoreType.DMA,
    ],
    compiler_params=pltpu.CompilerParams(
        needs_layout_passes=False, use_tc_tiling_on_sc=False),
)
def _sc_loss(table_hbm, idxi_hbm, idxj_hbm, out_hbm, *scratch):
    _body(table_hbm, idxi_hbm, idxj_hbm, out_hbm, *scratch)


def kernel(node_i, node_j, table):
    idx_i = node_i.astype(jnp.int32).reshape(_NW, _NCHUNK, _CHUNK)
    idx_j = node_j.astype(jnp.int32).reshape(_NW, _NCHUNK, _CHUNK)
    return _sc_loss(table, idx_i, idx_j)


# speedup vs baseline: 1.6554x; 1.6554x over previous
import functools

import jax
import jax.numpy as jnp
from jax import lax
from jax.experimental import pallas as pl
from jax.experimental.pallas import tpu as pltpu
from jax.experimental.pallas import tpu_sc as plsc

_NUM_NODES = 1000000
_DIM = 64
_BATCH = 16384
_NC, _NS, _L = 2, 16, 16
_NW = _NC * _NS
_BPW = _BATCH // _NW
_PACK = 128 // _DIM

_LN2 = 0.6931471805599453
_SQRT2 = 1.4142135623730951


def _neg_log(p):
    bits = plsc.bitcast(p, jnp.int32)
    e = lax.shift_right_logical(bits, 23) - 127
    m = plsc.bitcast(
        (bits & jnp.int32(0x007FFFFF)) | jnp.int32(0x3F800000), jnp.float32
    )
    big = m > jnp.float32(_SQRT2)
    m = jnp.where(big, m * jnp.float32(0.5), m)
    e = jnp.where(big, e + 1, e)
    s = (m - jnp.float32(1.0)) / (m + jnp.float32(1.0))
    s2 = s * s
    lnm = s * (jnp.float32(2.0) + s2 * (jnp.float32(2.0 / 3.0)
               + s2 * (jnp.float32(2.0 / 5.0) + s2 * jnp.float32(2.0 / 7.0))))
    return -(e.astype(jnp.float32) * jnp.float32(_LN2) + lnm)


def _body(table_hbm, idxi_hbm, idxj_hbm, dummy_hbm, out_hbm,
          idxi_v, idxj_v, rows_i, rows_j, out_v, sem_i, sem_j):
    wid = lax.axis_index("s") * _NC + lax.axis_index("c")

    pltpu.sync_copy(idxi_hbm.at[wid], idxi_v)
    pltpu.sync_copy(idxj_hbm.at[wid], idxj_v)

    def fire(b, carry):
        vi16 = idxi_v[pl.ds(b * _L, _L)]
        vj16 = idxj_v[pl.ds(b * _L, _L)]
        for l in range(_L):
            r2 = b * (_L // _PACK) + l // _PACK
            c2 = (l % _PACK) * _DIM
            pltpu.async_copy(
                table_hbm.at[vi16[l]], rows_i.at[r2, pl.ds(c2, _DIM)], sem_i)
            pltpu.async_copy(
                table_hbm.at[vj16[l]], rows_j.at[r2, pl.ds(c2, _DIM)], sem_j)
        return carry

    lax.fori_loop(0, _BPW // _L, fire, None)
    pltpu.make_async_copy(dummy_hbm, rows_i, sem_i).wait()
    pltpu.make_async_copy(dummy_hbm, rows_j, sem_j).wait()

    lanes = lax.iota(jnp.int32, _L)
    par = (lanes & 1) * _DIM

    def group(g, carry):
        row2 = lax.shift_right_logical(lanes + g * _L, 1)
        acc = jnp.zeros((_L,), jnp.float32)
        for d in range(_DIM):
            col = par + d
            vi = plsc.load_gather(rows_i, [row2, col])
            vj = plsc.load_gather(rows_j, [row2, col])
            acc = acc + vi * vj
        p = jnp.float32(1.0) / (jnp.float32(1.0) + jnp.exp(-acc))
        p = jnp.maximum(p, jnp.float32(1e-8))
        out_v[pl.ds(g * _L, _L)] = _neg_log(p)
        return carry

    lax.fori_loop(0, _BPW // _L, group, None)
    pltpu.sync_copy(out_v, out_hbm.at[pl.ds(wid * _BPW, _BPW)])


@functools.partial(
    pl.kernel,
    out_type=jax.ShapeDtypeStruct((_BATCH,), jnp.float32),
    mesh=plsc.VectorSubcoreMesh(
        core_axis_name="c", subcore_axis_name="s",
        num_cores=_NC, num_subcores=_NS),
    scratch_types=[
        pltpu.VMEM((_BPW,), jnp.int32),
        pltpu.VMEM((_BPW,), jnp.int32),
        pltpu.VMEM((_BPW // _PACK, 128), jnp.float32),
        pltpu.VMEM((_BPW // _PACK, 128), jnp.float32),
        pltpu.VMEM((_BPW,), jnp.float32),
        pltpu.SemaphoreType.DMA,
        pltpu.SemaphoreType.DMA,
    ],
    compiler_params=pltpu.CompilerParams(needs_layout_passes=False),
)
def _sc_loss(table_hbm, idxi_hbm, idxj_hbm, dummy_hbm, out_hbm, *scratch):
    _body(table_hbm, idxi_hbm, idxj_hbm, dummy_hbm, out_hbm, *scratch)


def kernel(node_i, node_j, table):
    idx_i = node_i.astype(jnp.int32).reshape(_NW, _BPW)
    idx_j = node_j.astype(jnp.int32).reshape(_NW, _BPW)
    dummy = jnp.zeros((_BPW // _PACK, 128), jnp.float32)
    return _sc_loss(table, idx_i, idx_j, dummy)
